# baseline (speedup 1.0000x reference)
import functools

import jax
import jax.numpy as jnp
from jax.experimental import pallas as pl
from jax.experimental.pallas import tpu as pltpu

NUM_ATOMS = 1024
HID = 768
EMB = 768
ANT_LEN = 4
BATCH = 1024

BB = 256


def _body(rep_ref, mask_ref, wih_ref, whh_ref, bih_ref, bhh_ref,
          hw_ref, hb_ref, emb_ref, out_ref):
    rep = rep_ref[...]
    mask = mask_ref[...]
    wih = wih_ref[...]
    whh = whh_ref[...]
    b_ih = bih_ref[...]
    b_hh = bhh_ref[...]
    hw = hw_ref[...]
    hb = hb_ref[...]
    emb = emb_ref[...]

    def mm_t(a, b):
        return jax.lax.dot_general(a, b, (((1,), (1,)), ((), ())),
                                   preferred_element_type=jnp.float32)

    n_iota = jax.lax.broadcasted_iota(jnp.int32, (1, NUM_ATOMS), 1)
    col0 = n_iota == 0
    neg_inf = jnp.float32(-jnp.inf)

    half = rep.shape[0] // 2
    reps = [rep[:half], rep[half:]]
    masks = [mask[:half], mask[half:]]
    hs = [jnp.zeros((half, EMB), jnp.float32) for _ in range(2)]
    curs = [reps[0], reps[1]]
    prev_inds = [None, None]

    for j in range(ANT_LEN):
        gis = [mm_t(curs[k], wih) + b_ih for k in range(2)]
        ghs = [mm_t(hs[k], whh) + b_hh for k in range(2)]
        logit_list = []
        for k in range(2):
            gi, gh = gis[k], ghs[k]
            r = jax.nn.sigmoid(gi[:, :EMB] + gh[:, :EMB])
            z = jax.nn.sigmoid(gi[:, EMB:2 * EMB] + gh[:, EMB:2 * EMB])
            n = jnp.tanh(gi[:, 2 * EMB:] + r * gh[:, 2 * EMB:])
            hs[k] = (1.0 - z) * n + z * hs[k]
            logit_list.append(mm_t(hs[k], hw) + hb)
        for k in range(2):
            logits = logit_list[k]
            mask_k = masks[k]
            if j == 0:
                empty = jnp.sum(mask_k, axis=-1, keepdims=True) == 0.0
                mask_k = jnp.where(col0 & empty, 1.0, mask_k)
            else:
                mask_k = jnp.where(prev_inds[k] == 0, 0.0, mask_k)
                mask_k = jnp.where(col0, 1.0, mask_k)

            masked = jnp.where(mask_k != 0.0, logits, neg_inf)
            mx = jnp.max(masked, axis=-1, keepdims=True)
            cand = jnp.where(masked == mx, n_iota, NUM_ATOMS)
            ind = jnp.min(cand, axis=-1, keepdims=True)
            sel = n_iota == ind
            onehot = sel.astype(jnp.float32)
            out_ref[pl.ds(k * half, half), j, :] = onehot
            masks[k] = jnp.where(sel, 0.0, mask_k)
            prev_inds[k] = ind

            if j + 1 < ANT_LEN:
                wsum = jnp.dot(onehot, emb,
                               preferred_element_type=jnp.float32)
                curs[k] = reps[k] + wsum


@jax.jit
def _run(rep, x_, wih_t, whh_t, b_ih, b_hh, hw_t, hb, emb):
    grid = (BATCH // BB,)
    const = lambda i: (0, 0)
    return pl.pallas_call(
        _body,
        grid=grid,
        in_specs=[
            pl.BlockSpec((BB, HID), lambda i: (i, 0)),
            pl.BlockSpec((BB, NUM_ATOMS), lambda i: (i, 0)),
            pl.BlockSpec((3 * EMB, HID), const),
            pl.BlockSpec((3 * EMB, EMB), const),
            pl.BlockSpec((1, 3 * EMB), const),
            pl.BlockSpec((1, 3 * EMB), const),
            pl.BlockSpec((NUM_ATOMS, EMB), const),
            pl.BlockSpec((1, NUM_ATOMS), const),
            pl.BlockSpec((NUM_ATOMS, EMB), const),
        ],
        out_specs=pl.BlockSpec((BB, ANT_LEN, NUM_ATOMS), lambda i: (i, 0, 0)),
        out_shape=jax.ShapeDtypeStruct((BATCH, ANT_LEN, NUM_ATOMS), jnp.float32),
        compiler_params=pltpu.CompilerParams(
            dimension_semantics=("parallel",)),
    )(rep, x_, wih_t, whh_t, b_ih, b_hh, hw_t, hb, emb)


def kernel(representation_emb, x_, W_ih, W_hh, b_ih, b_hh, head_w, head_b,
           atom_embedding):
    return _run(representation_emb, x_,
                W_ih, W_hh,
                b_ih.reshape(1, -1), b_hh.reshape(1, -1),
                head_w, head_b.reshape(1, -1),
                atom_embedding)


# speedup vs baseline: 1.6275x; 1.6275x over previous
import functools

import jax
import jax.numpy as jnp
from jax.experimental import pallas as pl
from jax.experimental.pallas import tpu as pltpu

NUM_ATOMS = 1024
HID = 768
EMB = 768
ANT_LEN = 4
BATCH = 1024

BB = 256


def _body(rep_ref, mask_ref, wih_ref, whh_ref, bih_ref, bhh_ref,
          hw_ref, hb_ref, emb_ref, out_ref):
    rep = rep_ref[...]
    mask = mask_ref[...]
    wih = wih_ref[...]
    whh = whh_ref[...]
    b_ih = bih_ref[...]
    b_hh = bhh_ref[...]
    hw = hw_ref[...]
    hb = hb_ref[...]
    emb = emb_ref[...]

    def mm_t(a, b):
        return jax.lax.dot_general(a, b, (((1,), (1,)), ((), ())),
                                   preferred_element_type=jnp.float32)

    n_iota = jax.lax.broadcasted_iota(jnp.int32, (1, NUM_ATOMS), 1)
    col0 = n_iota == 0
    neg_inf = jnp.float32(-jnp.inf)

    h = jnp.zeros((rep.shape[0], EMB), dtype=jnp.float32)
    cur = rep
    prev_ind = None
    for j in range(ANT_LEN):
        gi = mm_t(cur, wih) + b_ih
        gh = mm_t(h, whh) + b_hh
        r = jax.nn.sigmoid(gi[:, :EMB] + gh[:, :EMB])
        z = jax.nn.sigmoid(gi[:, EMB:2 * EMB] + gh[:, EMB:2 * EMB])
        n = jnp.tanh(gi[:, 2 * EMB:] + r * gh[:, 2 * EMB:])
        h = (1.0 - z) * n + z * h

        logits = mm_t(h, hw) + hb

        if j == 0:
            empty = jnp.sum(mask, axis=-1, keepdims=True) == 0.0
            mask = jnp.where(col0 & empty, 1.0, mask)
        else:
            mask = jnp.where(prev_ind == 0, 0.0, mask)
            mask = jnp.where(col0, 1.0, mask)

        masked = jnp.where(mask != 0.0, logits, neg_inf)
        mx = jnp.max(masked, axis=-1, keepdims=True)
        cand = jnp.where(masked == mx, n_iota, NUM_ATOMS)
        ind = jnp.min(cand, axis=-1, keepdims=True)
        sel = n_iota == ind
        onehot = sel.astype(jnp.float32)
        out_ref[:, j, :] = onehot
        mask = jnp.where(sel, 0.0, mask)
        prev_ind = ind

        if j + 1 < ANT_LEN:
            wsum = jnp.dot(onehot, emb, preferred_element_type=jnp.float32)
            cur = rep + wsum


@jax.jit
def _run(rep, x_, wih_t, whh_t, b_ih, b_hh, hw_t, hb, emb):
    grid = (BATCH // BB,)
    const = lambda i: (0, 0)
    return pl.pallas_call(
        _body,
        grid=grid,
        in_specs=[
            pl.BlockSpec((BB, HID), lambda i: (i, 0)),
            pl.BlockSpec((BB, NUM_ATOMS), lambda i: (i, 0)),
            pl.BlockSpec((3 * EMB, HID), const),
            pl.BlockSpec((3 * EMB, EMB), const),
            pl.BlockSpec((1, 3 * EMB), const),
            pl.BlockSpec((1, 3 * EMB), const),
            pl.BlockSpec((NUM_ATOMS, EMB), const),
            pl.BlockSpec((1, NUM_ATOMS), const),
            pl.BlockSpec((NUM_ATOMS, EMB), const),
        ],
        out_specs=pl.BlockSpec((BB, ANT_LEN, NUM_ATOMS), lambda i: (i, 0, 0)),
        out_shape=jax.ShapeDtypeStruct((BATCH, ANT_LEN, NUM_ATOMS), jnp.float32),
        compiler_params=pltpu.CompilerParams(
            dimension_semantics=("parallel",)),
    )(rep, x_, wih_t, whh_t, b_ih, b_hh, hw_t, hb, emb)


def kernel(representation_emb, x_, W_ih, W_hh, b_ih, b_hh, head_w, head_b,
           atom_embedding):
    return _run(representation_emb, x_,
                W_ih, W_hh,
                b_ih.reshape(1, -1), b_hh.reshape(1, -1),
                head_w, head_b.reshape(1, -1),
                atom_embedding)


# speedup vs baseline: 1.8813x; 1.1559x over previous
import functools

import jax
import jax.numpy as jnp
from jax.experimental import pallas as pl
from jax.experimental.pallas import tpu as pltpu

NUM_ATOMS = 1024
HID = 768
EMB = 768
ANT_LEN = 4
BATCH = 1024

BB = 256


def _body(rep_ref, mask_ref, wih_ref, whh_ref, bih_ref, bhh_ref,
          hw_ref, hb_ref, emb_ref, out_ref):
    rep = rep_ref[...]
    mask = mask_ref[...]
    wih = wih_ref[...]
    whh = whh_ref[...]
    b_ih = bih_ref[...]
    b_hh = bhh_ref[...]
    hw = hw_ref[...]
    hb = hb_ref[...]
    emb = emb_ref[...]

    def mm_t(a, b):
        return jax.lax.dot_general(a, b, (((1,), (1,)), ((), ())),
                                   preferred_element_type=jnp.float32)

    n_iota = jax.lax.broadcasted_iota(jnp.int32, (1, NUM_ATOMS), 1)
    col0 = n_iota == 0
    neg_inf = jnp.float32(-jnp.inf)

    gi = mm_t(rep, wih) + b_ih
    gh = b_hh
    prev_ind = None
    h = None
    for j in range(ANT_LEN):
        r = jax.nn.sigmoid(gi[:, :EMB] + gh[:, :EMB])
        z = jax.nn.sigmoid(gi[:, EMB:2 * EMB] + gh[:, EMB:2 * EMB])
        n = jnp.tanh(gi[:, 2 * EMB:] + r * gh[:, 2 * EMB:])
        h = (1.0 - z) * n if j == 0 else (1.0 - z) * n + z * h

        logits = mm_t(h, hw) + hb

        if j + 1 < ANT_LEN:
            gh = mm_t(h, whh) + b_hh

        if j == 0:
            empty = jnp.sum(mask, axis=-1, keepdims=True) == 0.0
            mask = jnp.where(col0 & empty, 1.0, mask)
        else:
            mask = jnp.where(prev_ind == 0, 0.0, mask)
            mask = jnp.where(col0, 1.0, mask)

        masked = jnp.where(mask != 0.0, logits, neg_inf)
        mx = jnp.max(masked, axis=-1, keepdims=True)
        cand = jnp.where(masked == mx, n_iota, NUM_ATOMS)
        ind = jnp.min(cand, axis=-1, keepdims=True)
        sel = n_iota == ind
        onehot = sel.astype(jnp.float32)
        out_ref[:, j, :] = onehot
        mask = jnp.where(sel, 0.0, mask)
        prev_ind = ind

        if j + 1 < ANT_LEN:
            wsum = jnp.dot(onehot, emb, preferred_element_type=jnp.float32)
            gi = mm_t(rep + wsum, wih) + b_ih


@jax.jit
def _run(rep, x_, wih_t, whh_t, b_ih, b_hh, hw_t, hb, emb):
    grid = (BATCH // BB,)
    const = lambda i: (0, 0)
    return pl.pallas_call(
        _body,
        grid=grid,
        in_specs=[
            pl.BlockSpec((BB, HID), lambda i: (i, 0)),
            pl.BlockSpec((BB, NUM_ATOMS), lambda i: (i, 0)),
            pl.BlockSpec((3 * EMB, HID), const),
            pl.BlockSpec((3 * EMB, EMB), const),
            pl.BlockSpec((1, 3 * EMB), const),
            pl.BlockSpec((1, 3 * EMB), const),
            pl.BlockSpec((NUM_ATOMS, EMB), const),
            pl.BlockSpec((1, NUM_ATOMS), const),
            pl.BlockSpec((NUM_ATOMS, EMB), const),
        ],
        out_specs=pl.BlockSpec((BB, ANT_LEN, NUM_ATOMS), lambda i: (i, 0, 0)),
        out_shape=jax.ShapeDtypeStruct((BATCH, ANT_LEN, NUM_ATOMS), jnp.float32),
        compiler_params=pltpu.CompilerParams(
            dimension_semantics=("parallel",)),
    )(rep, x_, wih_t, whh_t, b_ih, b_hh, hw_t, hb, emb)


def kernel(representation_emb, x_, W_ih, W_hh, b_ih, b_hh, head_w, head_b,
           atom_embedding):
    return _run(representation_emb, x_,
                W_ih, W_hh,
                b_ih.reshape(1, -1), b_hh.reshape(1, -1),
                head_w, head_b.reshape(1, -1),
                atom_embedding)
